# baseline (speedup 1.0000x reference)
import jax
import jax.numpy as jnp
from jax import lax
from jax.experimental import pallas as pl
from jax.experimental.pallas import tpu as pltpu
from jax.experimental.pallas import tpu_sc as plsc

B, DIM, H, W_SP = 64, 768, 24, 24
NUM_EXPERTS = 16
HW = H * W_SP

NB_SC = 32
NB_TC = B - NB_SC
CC = 128
JGB = 96
NEG_BIG = -3.0e38


def _router_body(x_ref, w_ref, b_ref, scores_ref, probs_ref, idx_ref):
    bb = x_ref.shape[0]
    for i in range(bb):
        logits = jnp.dot(w_ref[...], x_ref[i],
                         preferred_element_type=jnp.float32)
        logits = logits + b_ref[0, :][:, None]
        m = jnp.max(logits, axis=0, keepdims=True)
        e = jnp.exp(logits - m)
        s = jnp.sum(e, axis=0, keepdims=True)
        scores = e / s
        scores_ref[i] = scores

        lane = jax.lax.broadcasted_iota(jnp.int32, logits.shape, 0)
        i1 = jnp.argmax(logits, axis=0).astype(jnp.int32)
        masked = jnp.where(lane == i1[None, :], -jnp.inf, logits)
        i2 = jnp.argmax(masked, axis=0).astype(jnp.int32)
        v1 = jnp.max(scores, axis=0)
        v2 = jnp.max(jnp.where(lane == i1[None, :], -jnp.inf, scores), axis=0)
        t = v1 + v2
        probs_ref[i] = jnp.stack([v1 / t, v2 / t], axis=0)
        idx_ref[i] = jnp.stack([i1, i2], axis=0)


def _tc_call(xr, W, b):
    bb = 4
    grid = (NB_TC // bb,)
    return pl.pallas_call(
        _router_body,
        grid=grid,
        in_specs=[
            pl.BlockSpec((bb, DIM, HW), lambda i: (i, 0, 0)),
            pl.BlockSpec((NUM_EXPERTS, DIM), lambda i: (0, 0)),
            pl.BlockSpec((1, NUM_EXPERTS), lambda i: (0, 0)),
        ],
        out_specs=[
            pl.BlockSpec((bb, NUM_EXPERTS, HW), lambda i: (i, 0, 0)),
            pl.BlockSpec((bb, 2, HW), lambda i: (i, 0, 0)),
            pl.BlockSpec((bb, 2, HW), lambda i: (i, 0, 0)),
        ],
        out_shape=[
            jax.ShapeDtypeStruct((NB_TC, NUM_EXPERTS, HW), jnp.float32),
            jax.ShapeDtypeStruct((NB_TC, 2, HW), jnp.float32),
            jax.ShapeDtypeStruct((NB_TC, 2, HW), jnp.int32),
        ],
        compiler_params=pltpu.CompilerParams(
            dimension_semantics=("parallel",),
        ),
    )(xr, W, b.reshape(1, NUM_EXPERTS))


def _sc_task(x_hbm, wsp_hbm, bsp_vmem, xbuf, wspbuf, acc_vmem, sc_vmem,
             pr_vmem, ix_vmem, scores_hbm, probs_hbm, idx_hbm, batch):
    batch_x = batch + NB_TC
    e_iota = [jnp.full((16,), e, jnp.int32) for e in range(NUM_EXPERTS)]

    bspl = [bsp_vmem[e, :] for e in range(NUM_EXPERTS)]

    def init_body(jg, carry):
        for e in range(NUM_EXPERTS):
            acc_vmem[e, pl.ds(jg * 16, 16)] = bspl[e]
        return carry

    lax.fori_loop(0, HW // 16, init_body, 0)

    def chunk_body(cc, carry):
        pltpu.sync_copy(x_hbm.at[batch_x, pl.ds(cc * CC, CC), :], xbuf)
        pltpu.sync_copy(wsp_hbm.at[pl.ds(cc * CC, CC)], wspbuf)

        def round_body(i, carry2):
            c = i // (HW // 16)
            k = (i % (HW // 16)) * 16
            u = lax.bitcast_convert_type(xbuf[c, pl.ds(k, 16)], jnp.int32)
            odd = lax.shift_right_logical(u, 16) & 1
            t = (u + 0x7FFF) + odd
            r = t & jnp.int32(-65536)
            xbuf[c, pl.ds(k, 16)] = lax.bitcast_convert_type(r, jnp.float32)
            return carry2

        lax.fori_loop(0, CC * (HW // 16), round_body, 0)

        def jgb_body(jgb, carry2):
            p0 = jgb * JGB
            for eg in range(2):
                init = tuple(
                    acc_vmem[eg * 8 + e, pl.ds(p0 + j * 16, 16)]
                    for e in range(8) for j in range(6))

                def cbody(c, accs, _eg=eg, _p0=p0):
                    spl = [wspbuf[c, _eg * 8 + e, :] for e in range(8)]
                    xv = [xbuf[c, pl.ds(_p0 + j * 16, 16)] for j in range(6)]
                    return tuple(accs[e * 6 + j] + spl[e] * xv[j]
                                 for e in range(8) for j in range(6))

                accs = lax.fori_loop(0, CC, cbody, init)
                for e in range(8):
                    for j in range(6):
                        acc_vmem[eg * 8 + e, pl.ds(p0 + j * 16, 16)] = \
                            accs[e * 6 + j]
            return carry2

        lax.fori_loop(0, HW // JGB, jgb_body, 0)
        return carry

    lax.fori_loop(0, DIM // CC, chunk_body, 0)

    def ep_body(jg, carry):
        q = jg * 16
        a = [acc_vmem[e, pl.ds(q, 16)] for e in range(NUM_EXPERTS)]
        m = a[0]
        for e in range(1, NUM_EXPERTS):
            m = jnp.maximum(m, a[e])
        ex = [jnp.exp(a[e] - m) for e in range(NUM_EXPERTS)]
        s = ex[0]
        for e in range(1, NUM_EXPERTS):
            s = s + ex[e]
        rinv = 1.0 / s
        sc = [ex[e] * rinv for e in range(NUM_EXPERTS)]
        for e in range(NUM_EXPERTS):
            sc_vmem[e, pl.ds(q, 16)] = sc[e]
        v1, i1 = sc[0], e_iota[0]
        for e in range(1, NUM_EXPERTS):
            gt = sc[e] > v1
            v1 = jnp.where(gt, sc[e], v1)
            i1 = jnp.where(gt, e_iota[e], i1)
        v2 = jnp.where(i1 == e_iota[0], NEG_BIG, sc[0])
        i2 = e_iota[0]
        for e in range(1, NUM_EXPERTS):
            cand = jnp.where(i1 == e_iota[e], NEG_BIG, sc[e])
            gt = cand > v2
            v2 = jnp.where(gt, cand, v2)
            i2 = jnp.where(gt, e_iota[e], i2)
        t = 1.0 / (v1 + v2)
        pr_vmem[0, pl.ds(q, 16)] = v1 * t
        pr_vmem[1, pl.ds(q, 16)] = v2 * t
        ix_vmem[0, pl.ds(q, 16)] = i1
        ix_vmem[1, pl.ds(q, 16)] = i2
        return carry

    lax.fori_loop(0, HW // 16, ep_body, 0)

    pltpu.sync_copy(sc_vmem, scores_hbm.at[batch])
    pltpu.sync_copy(pr_vmem, probs_hbm.at[batch])
    pltpu.sync_copy(ix_vmem, idx_hbm.at[batch])


def _sc_router_body(x_hbm, wsp_hbm, bsp_hbm, scores_hbm, probs_hbm, idx_hbm,
                    bsp_vmem, xbuf, wspbuf, acc_vmem, sc_vmem, pr_vmem,
                    ix_vmem):
    c = lax.axis_index("c")
    s = lax.axis_index("s")
    wid = s * 2 + c
    pltpu.sync_copy(bsp_hbm, bsp_vmem)
    _sc_task(x_hbm, wsp_hbm, bsp_vmem, xbuf, wspbuf, acc_vmem, sc_vmem,
             pr_vmem, ix_vmem, scores_hbm, probs_hbm, idx_hbm, wid)


def _sc_call(xr, Wsp, bsp):
    f = pl.kernel(
        _sc_router_body,
        out_type=(
            jax.ShapeDtypeStruct((NB_SC, NUM_EXPERTS, HW), jnp.float32),
            jax.ShapeDtypeStruct((NB_SC, 2, HW), jnp.float32),
            jax.ShapeDtypeStruct((NB_SC, 2, HW), jnp.int32),
        ),
        mesh=plsc.VectorSubcoreMesh(core_axis_name="c", subcore_axis_name="s"),
        compiler_params=pltpu.CompilerParams(use_tc_tiling_on_sc=False),
        scratch_types=[
            pltpu.VMEM((NUM_EXPERTS, 16), jnp.float32),
            pltpu.VMEM((CC, HW), jnp.float32),
            pltpu.VMEM((CC, NUM_EXPERTS, 16), jnp.float32),
            pltpu.VMEM((NUM_EXPERTS, HW), jnp.float32),
            pltpu.VMEM((NUM_EXPERTS, HW), jnp.float32),
            pltpu.VMEM((2, HW), jnp.float32),
            pltpu.VMEM((2, HW), jnp.int32),
        ],
    )
    return f(xr, Wsp, bsp)


def kernel(x, W, b):
    bsz = x.shape[0]
    xr = x.reshape(bsz, DIM, HW)
    scores_tc, probs_tc, idx_tc = _tc_call(xr, W, b)
    wu = lax.bitcast_convert_type(W, jnp.int32)
    wodd = lax.shift_right_logical(wu, 16) & 1
    w_r = lax.bitcast_convert_type(((wu + 0x7FFF) + wodd) & jnp.int32(-65536),
                                   jnp.float32)
    wsp = jnp.broadcast_to(w_r.T[:, :, None], (DIM, NUM_EXPERTS, 16))
    bsp = jnp.broadcast_to(b[:, None], (NUM_EXPERTS, 16))
    scores_sc, probs_sc, idx_sc = _sc_call(xr, wsp, bsp)
    scores = jnp.concatenate([scores_tc, scores_sc], axis=0)
    probs = jnp.concatenate([probs_tc, probs_sc], axis=0)
    idx = jnp.concatenate([idx_tc, idx_sc], axis=0)
    return (probs.reshape(bsz, 2, H, W_SP),
            idx.reshape(bsz, 2, H, W_SP),
            scores.reshape(bsz, NUM_EXPERTS, H, W_SP))


# speedup vs baseline: 5.9047x; 5.9047x over previous
import jax
import jax.numpy as jnp
from jax.experimental import pallas as pl
from jax.experimental.pallas import tpu as pltpu

B, DIM, H, W_SP = 64, 768, 24, 24
NUM_EXPERTS = 16
HW = H * W_SP


def _router_body(x_ref, w_ref, b_ref, scores_ref, probs_ref, idx_ref):
    bb = x_ref.shape[0]
    for i in range(bb):
        logits = jnp.dot(w_ref[...], x_ref[i],
                         preferred_element_type=jnp.float32)
        logits = logits + b_ref[0, :][:, None]
        m = jnp.max(logits, axis=0, keepdims=True)
        e = jnp.exp(logits - m)
        s = jnp.sum(e, axis=0, keepdims=True)
        scores = e / s
        scores_ref[i] = scores

        lane = jax.lax.broadcasted_iota(jnp.int32, logits.shape, 0)
        i1 = jnp.argmax(logits, axis=0).astype(jnp.int32)
        masked = jnp.where(lane == i1[None, :], -jnp.inf, logits)
        i2 = jnp.argmax(masked, axis=0).astype(jnp.int32)
        v1 = jnp.max(scores, axis=0)
        v2 = jnp.max(jnp.where(lane == i1[None, :], -jnp.inf, scores), axis=0)
        t = v1 + v2
        probs_ref[i] = jnp.stack([v1 / t, v2 / t], axis=0)
        idx_ref[i] = jnp.stack([i1, i2], axis=0)


def kernel(x, W, b):
    bsz = x.shape[0]
    xr = x.reshape(bsz, DIM, HW)
    bb = 4
    grid = (bsz // bb,)
    scores, probs, idx = pl.pallas_call(
        _router_body,
        grid=grid,
        in_specs=[
            pl.BlockSpec((bb, DIM, HW), lambda i: (i, 0, 0)),
            pl.BlockSpec((NUM_EXPERTS, DIM), lambda i: (0, 0)),
            pl.BlockSpec((1, NUM_EXPERTS), lambda i: (0, 0)),
        ],
        out_specs=[
            pl.BlockSpec((bb, NUM_EXPERTS, HW), lambda i: (i, 0, 0)),
            pl.BlockSpec((bb, 2, HW), lambda i: (i, 0, 0)),
            pl.BlockSpec((bb, 2, HW), lambda i: (i, 0, 0)),
        ],
        out_shape=[
            jax.ShapeDtypeStruct((bsz, NUM_EXPERTS, HW), jnp.float32),
            jax.ShapeDtypeStruct((bsz, 2, HW), jnp.float32),
            jax.ShapeDtypeStruct((bsz, 2, HW), jnp.int32),
        ],
        compiler_params=pltpu.CompilerParams(
            dimension_semantics=("parallel",),
        ),
    )(xr, W, b.reshape(1, NUM_EXPERTS))
    return (probs.reshape(bsz, 2, H, W_SP),
            idx.reshape(bsz, 2, H, W_SP),
            scores.reshape(bsz, NUM_EXPERTS, H, W_SP))
